# Initial kernel scaffold; baseline (speedup 1.0000x reference)
import jax
import jax.numpy as jnp
from jax.experimental import pallas as pl


def kernel(x, edge_index, node_index, edge_set, W1, b1, W2, b2, Wr, br):
    raise NotImplementedError("write your pallas kernel here")


# speedup vs baseline: 5.2146x; 5.2146x over previous
import functools

import jax
import jax.numpy as jnp
from jax import lax
from jax.experimental import pallas as pl
from jax.experimental.pallas import tpu as pltpu
from jax.experimental.pallas import tpu_sc as plsc

N = 10000
NP = 10240
E = 160000
F = 256
HALF = 128
NT = 16
EPT = E // NT
CH = 80
NCH = EPT // CH
SL = NP // NT
ALPHA = 1000000.0
BLK = 1000
GRID = N // BLK

_mesh = plsc.VectorSubcoreMesh(core_axis_name="c", subcore_axis_name="s")


def _stats_body(src3, dst3, nidx_h, ones_h, dout_h, din_h, cnt_h, lastp_h,
                srcp_h, srcl, dstl, wvs, ones_v, nv, lastv_s, srcv_s, vbuf,
                hist_sp, cnt_sp, sem_h, sem_w):
    c = lax.axis_index("c")
    s = lax.axis_index("s")
    for j in range(SL // 16):
        vbuf[pl.ds(16 * j, 16)] = jnp.zeros((16,), jnp.float32)
    pltpu.sync_copy(vbuf, hist_sp.at[pl.ds(s * SL, SL)])
    pltpu.sync_copy(src3.at[s], srcl)
    pltpu.sync_copy(ones_h, ones_v)

    @pl.when(c == 1)
    def _():
        pltpu.sync_copy(vbuf, cnt_sp.at[pl.ds(s * SL, SL)])
        pltpu.sync_copy(dst3.at[s], dstl)
        pltpu.sync_copy(nidx_h, nv)

    plsc.subcore_barrier()

    @pl.when(c == 0)
    def _():
        def bk(k, _):
            pltpu.async_copy(ones_v, hist_sp.at[srcl.at[k]], sem_h, add=True)

            @pl.when(k >= 4)
            def _():
                pltpu.make_async_copy(ones_v, hist_sp.at[srcl.at[0]],
                                      sem_h).wait()
            return 0

        lax.fori_loop(0, NCH, bk, 0)
        for _ in range(4):
            pltpu.make_async_copy(ones_v, hist_sp.at[srcl.at[0]], sem_h).wait()

    @pl.when(c == 1)
    def _():
        nvec = nv[...]
        lanes = lax.iota(jnp.int32, 16)
        base = s * EPT

        def bk(k, carry):
            lastv, srcv = carry
            for j in range(CH // 16):
                s16 = srcl[k, pl.ds(16 * j, 16)]
                d16 = dstl[k, pl.ds(16 * j, 16)]
                m = d16 == nvec
                w = jnp.where(m & (s16 < N - 1), 1.0, 0.0).astype(jnp.float32)
                wvs[k, pl.ds(16 * j, 16)] = w
                eid = base + k * CH + 16 * j + lanes
                upd = m & (eid > lastv)
                lastv = jnp.where(upd, eid, lastv)
                srcv = jnp.where(upd, s16, srcv)
            pltpu.async_copy(ones_v, hist_sp.at[dstl.at[k]], sem_h, add=True)
            pltpu.async_copy(wvs.at[k], cnt_sp.at[srcl.at[k]], sem_w, add=True)

            @pl.when(k >= 4)
            def _():
                pltpu.make_async_copy(ones_v, hist_sp.at[dstl.at[0]],
                                      sem_h).wait()
                pltpu.make_async_copy(wvs.at[0], cnt_sp.at[srcl.at[0]],
                                      sem_w).wait()
            return (lastv, srcv)

        init = (jnp.full((16,), -1, jnp.int32), jnp.full((16,), -1, jnp.int32))
        lastv, srcv = lax.fori_loop(0, NCH, bk, init)
        for _ in range(4):
            pltpu.make_async_copy(ones_v, hist_sp.at[dstl.at[0]], sem_h).wait()
            pltpu.make_async_copy(wvs.at[0], cnt_sp.at[srcl.at[0]],
                                  sem_w).wait()
        lastv_s[...] = lastv
        srcv_s[...] = srcv
        pltpu.sync_copy(lastv_s, lastp_h.at[s])
        pltpu.sync_copy(srcv_s, srcp_h.at[s])

    plsc.subcore_barrier()

    @pl.when(c == 0)
    def _():
        pltpu.sync_copy(hist_sp.at[pl.ds(s * SL, SL)], vbuf)
        pltpu.sync_copy(vbuf, dout_h.at[pl.ds(s * SL, SL)])

    @pl.when(c == 1)
    def _():
        pltpu.sync_copy(hist_sp.at[pl.ds(s * SL, SL)], vbuf)
        pltpu.sync_copy(vbuf, din_h.at[pl.ds(s * SL, SL)])
        pltpu.sync_copy(cnt_sp.at[pl.ds(s * SL, SL)], vbuf)
        pltpu.sync_copy(vbuf, cnt_h.at[pl.ds(s * SL, SL)])


_sc_params = pltpu.CompilerParams(use_tc_tiling_on_sc=False)

_stats_call = functools.partial(
    pl.kernel, _stats_body, mesh=_mesh, compiler_params=_sc_params,
    out_type=(
        jax.ShapeDtypeStruct((NP,), jnp.float32),
        jax.ShapeDtypeStruct((NP,), jnp.float32),
        jax.ShapeDtypeStruct((NP,), jnp.float32),
        jax.ShapeDtypeStruct((NT, 16), jnp.int32),
        jax.ShapeDtypeStruct((NT, 16), jnp.int32),
    ),
    scratch_types=[
        pltpu.VMEM((NCH, CH), jnp.int32),
        pltpu.VMEM((NCH, CH), jnp.int32),
        pltpu.VMEM((NCH, CH), jnp.float32),
        pltpu.VMEM((CH,), jnp.float32),
        pltpu.VMEM((16,), jnp.int32),
        pltpu.VMEM((16,), jnp.int32),
        pltpu.VMEM((16,), jnp.int32),
        pltpu.VMEM((SL,), jnp.float32),
        pltpu.VMEM_SHARED((NP,), jnp.float32),
        pltpu.VMEM_SHARED((NP,), jnp.float32),
        pltpu.SemaphoreType.DMA,
        pltpu.SemaphoreType.DMA,
    ],
)


SLAB = 64


def _spmm_body(feat_h, src3, dst3, z2_h, agg_h, srcl, dstl, rows0, rows1,
               agg_sp, sem_g, sem_s):
    c = lax.axis_index("c")
    s = lax.axis_index("s")
    pltpu.sync_copy(src3.at[s], srcl)
    pltpu.sync_copy(dst3.at[s], dstl)

    for p in range(2):
        slab = c * 2 + p
        pltpu.sync_copy(z2_h, rows0)
        for j in range(SL // CH):
            pltpu.sync_copy(rows0, agg_sp.at[pl.ds(s * SL + CH * j, CH)])
        plsc.subcore_barrier()

        fview = feat_h.at[slab]

        def g_start(k, rbuf):
            pltpu.async_copy(fview.at[srcl.at[k]], rbuf, sem_g)

        def g_wait(rbuf):
            pltpu.make_async_copy(fview.at[srcl.at[0]], rbuf, sem_g).wait()

        def s_start(k, rbuf):
            pltpu.async_copy(rbuf, agg_sp.at[dstl.at[k]], sem_s, add=True)

        def s_wait(rbuf):
            pltpu.make_async_copy(rbuf, agg_sp.at[dstl.at[0]], sem_s).wait()

        g_start(0, rows0)

        def body_k(i, _):
            kk = 2 * i

            @pl.when(kk > 0)
            def _():
                s_wait(rows1)
            g_start(kk + 1, rows1)
            g_wait(rows0)
            s_start(kk, rows0)
            s_wait(rows0)
            g_start(kk + 2, rows0)
            g_wait(rows1)
            s_start(kk + 1, rows1)
            return 0

        lax.fori_loop(0, (NCH - 1) // 2, body_k, 0)
        s_wait(rows1)
        g_wait(rows0)
        s_start(NCH - 1, rows0)
        s_wait(rows0)
        plsc.subcore_barrier()
        for j in range(SL // CH):
            pltpu.sync_copy(agg_sp.at[pl.ds(s * SL + CH * j, CH)], rows0)
            pltpu.sync_copy(rows0, agg_h.at[slab, pl.ds(s * SL + CH * j, CH)])
        plsc.subcore_barrier()


_spmm_call = functools.partial(
    pl.kernel, _spmm_body, mesh=_mesh, compiler_params=_sc_params,
    out_type=jax.ShapeDtypeStruct((4, NP, SLAB), jnp.float32),
    scratch_types=[
        pltpu.VMEM((NCH, CH), jnp.int32),
        pltpu.VMEM((NCH, CH), jnp.int32),
        pltpu.VMEM((CH, SLAB), jnp.float32),
        pltpu.VMEM((CH, SLAB), jnp.float32),
        pltpu.VMEM_SHARED((NP, SLAB), jnp.float32),
        pltpu.SemaphoreType.DMA,
        pltpu.SemaphoreType.DMA,
    ],
)


def _prep_body(x_ref, dop_ref, dip_ref, feat_ref, doutr_ref, dinr_ref):
    dor = lax.rsqrt(jnp.maximum(dop_ref[...], 1.0))
    dir_ = lax.rsqrt(jnp.maximum(dip_ref[...], 1.0))
    xf = x_ref[...] * dor
    for j in range(4):
        feat_ref[j] = xf[:, SLAB * j:SLAB * (j + 1)]
    doutr_ref[...] = dor
    dinr_ref[...] = dir_


def _prep(x, dout, din):
    return pl.pallas_call(
        _prep_body,
        grid=(GRID,),
        in_specs=[
            pl.BlockSpec((BLK, F), lambda i: (i, 0)),
            pl.BlockSpec((BLK, 1), lambda i: (i, 0)),
            pl.BlockSpec((BLK, 1), lambda i: (i, 0)),
        ],
        out_specs=[
            pl.BlockSpec((4, BLK, SLAB), lambda i: (0, i, 0)),
            pl.BlockSpec((BLK, 1), lambda i: (i, 0)),
            pl.BlockSpec((BLK, 1), lambda i: (i, 0)),
        ],
        out_shape=[
            jax.ShapeDtypeStruct((4, NP, SLAB), jnp.float32),
            jax.ShapeDtypeStruct((N, 1), jnp.float32),
            jax.ShapeDtypeStruct((N, 1), jnp.float32),
        ],
    )(x, dout, din)


def _dense1_body(agg_ref, dinr_ref, doutr_ref, w_ref, b_ref, out_ref):
    z = b_ref[...]
    for j in range(4):
        z = z + jnp.dot(agg_ref[j] * dinr_ref[...], w_ref[j],
                        preferred_element_type=jnp.float32)
    h = jnp.where(z >= 0, z, 0.01 * z) * doutr_ref[...]
    for j in range(4):
        out_ref[j] = h[:, SLAB * j:SLAB * (j + 1)]


def _dense1(agg, dinr, doutr, w, b):
    return pl.pallas_call(
        _dense1_body,
        grid=(GRID,),
        in_specs=[
            pl.BlockSpec((4, BLK, SLAB), lambda i: (0, i, 0)),
            pl.BlockSpec((BLK, 1), lambda i: (i, 0)),
            pl.BlockSpec((BLK, 1), lambda i: (i, 0)),
            pl.BlockSpec((4, SLAB, F), lambda i: (0, 0, 0)),
            pl.BlockSpec((1, F), lambda i: (0, 0)),
        ],
        out_specs=pl.BlockSpec((4, BLK, SLAB), lambda i: (0, i, 0)),
        out_shape=jax.ShapeDtypeStruct((4, NP, SLAB), jnp.float32),
    )(agg, dinr, doutr, w, b)


def _dense2_body(agg_ref, dinr_ref, w_ref, b_ref, wr_ref, out_ref):
    z = b_ref[...]
    for j in range(4):
        z = z + jnp.dot(agg_ref[j] * dinr_ref[...], w_ref[j],
                        preferred_element_type=jnp.float32)
    h = jnp.where(z >= 0, z, 0.01 * z)
    out_ref[...] = jnp.dot(h, wr_ref[...], preferred_element_type=jnp.float32)


def _dense2(agg, dinr, w, b, wr):
    return pl.pallas_call(
        _dense2_body,
        grid=(GRID,),
        in_specs=[
            pl.BlockSpec((4, BLK, SLAB), lambda i: (0, i, 0)),
            pl.BlockSpec((BLK, 1), lambda i: (i, 0)),
            pl.BlockSpec((4, SLAB, F), lambda i: (0, 0, 0)),
            pl.BlockSpec((1, F), lambda i: (0, 0)),
            pl.BlockSpec((F, 2), lambda i: (0, 0)),
        ],
        out_specs=pl.BlockSpec((BLK, 2), lambda i: (i, 0)),
        out_shape=jax.ShapeDtypeStruct((N, 2), jnp.float32),
    )(agg, dinr, w, b, wr)


ROWS = 79
PAD = ROWS * HALF


def _sample_body(s_ref, q_ref, g_ref, cnt_ref, lastp_ref, srcp_ref, nidx_ref,
                 xlast_ref, wrb_ref, br_ref, oh_ref, lp_ref):
    r = lax.broadcasted_iota(jnp.int32, (ROWS, HALF), 0)
    col = lax.broadcasted_iota(jnp.int32, (ROWS, HALF), 1)
    i2 = r * HALF + col
    valid = (i2 >= 8) & (i2 <= N - 2)
    lastm = jnp.max(lastp_ref[...])
    srcl = jnp.sum(jnp.where(lastp_ref[...] == lastm, srcp_ref[...], 0))
    srcl = jnp.where(lastm < 0, -1, srcl)
    cnt = cnt_ref[...] - jnp.where(i2 == srcl, 1.0, 0.0)
    nm = jnp.where(cnt > 0.5, ALPHA, 0.0)
    nidx = jnp.sum(nidx_ref[...])
    csc = (jnp.sum(jnp.where(i2 == nidx, q_ref[...], 0.0))
           + jnp.sum(xlast_ref[...] * wrb_ref[...]) + jnp.sum(br_ref[...]))
    fd = jnp.where(valid, s_ref[...] + csc + nm, -1e30)
    mx = jnp.max(fd)
    e = jnp.where(valid, jnp.exp(fd - mx), 0.0)
    p = e / jnp.sum(e)
    lp = jnp.log(p)
    t = lp + g_ref[...]
    tm = jnp.max(t)
    idxn = jnp.min(jnp.where(t == tm, i2, jnp.int32(2**30)))
    sel = i2 == idxn
    oh_ref[...] = jnp.where(sel, 1.0, 0.0)
    lpv = jnp.sum(jnp.where(sel, jnp.where(valid, lp, 0.0), 0.0))
    lp_ref[...] = jnp.reshape(lpv, (1, 1))


def _sample(s_pad, q_pad, g_node, cnt_pad, lastp, srcp, nidx, xlast, wrb, br2):
    return pl.pallas_call(
        _sample_body,
        out_shape=[
            jax.ShapeDtypeStruct((ROWS, HALF), jnp.float32),
            jax.ShapeDtypeStruct((1, 1), jnp.float32),
        ],
    )(s_pad, q_pad, g_node, cnt_pad, lastp, srcp, nidx, xlast, wrb, br2)


def kernel(x, edge_index, node_index, edge_set, W1, b1, W2, b2, Wr, br):
    f32 = jnp.float32
    src = edge_index[0].astype(jnp.int32)
    dst = edge_index[1].astype(jnp.int32)
    src3 = src.reshape(NT, NCH, CH)
    dst3 = dst.reshape(NT, NCH, CH)
    nidx16 = jnp.full((16,), node_index, jnp.int32)
    ones80 = jnp.ones((CH,), f32)

    dout_h, din_h, cnt_h, lastp, srcp = _stats_call()(
        src3, dst3, nidx16, ones80)

    feat, doutr, dinr = _prep(x, dout_h[:N].reshape(N, 1),
                              din_h[:N].reshape(N, 1))
    w1r = W1.reshape(4, SLAB, F)
    w2r = W2.reshape(4, SLAB, F)
    z2 = jnp.zeros((CH, SLAB), f32)
    agg1 = _spmm_call()(feat, src3, dst3, z2)
    feat2 = _dense1(agg1, dinr, doutr, w1r, b1.reshape(1, F))
    agg2 = _spmm_call()(feat2, src3, dst3, z2)
    wr_ac = jnp.concatenate([Wr[0:F], Wr[2 * F:3 * F]], axis=1)
    sq = _dense2(agg2, dinr, w2r, b2.reshape(1, F), wr_ac)

    zpad = jnp.zeros((PAD - N,), f32)
    s_pad = jnp.concatenate([sq[:, 0], zpad]).reshape(ROWS, HALF)
    q_pad = jnp.concatenate([sq[:, 1], zpad]).reshape(ROWS, HALF)
    cnt_pad = cnt_h[:PAD].reshape(ROWS, HALF)
    g = jax.random.gumbel(jax.random.key(42), (N - 1 - 8,), f32)
    g_node = jnp.concatenate(
        [jnp.zeros((8,), f32), g, jnp.zeros((PAD - (N - 1),), f32)]
    ).reshape(ROWS, HALF)
    nidx11 = jnp.asarray(node_index, jnp.int32).reshape(1, 1)
    xlast = x[-1].reshape(2, HALF)
    wrb = Wr[F:2 * F, 0].reshape(2, HALF)
    br2 = br.reshape(1, 1)

    oh, lp = _sample(s_pad, q_pad, g_node, cnt_pad, lastp, srcp, nidx11,
                     xlast, wrb, br2)
    sample_full = oh.reshape(PAD)[:N - 1]
    log_prob = lp.reshape(())
    return (sample_full, log_prob)


# speedup vs baseline: 6.4101x; 1.2293x over previous
import functools

import jax
import jax.numpy as jnp
from jax import lax
from jax.experimental import pallas as pl
from jax.experimental.pallas import tpu as pltpu
from jax.experimental.pallas import tpu_sc as plsc

N = 10000
NP = 10240
E = 160000
F = 256
HALF = 128
NT = 16
EPT = E // NT
CH = 80
NCH = EPT // CH
SL = NP // NT
ALPHA = 1000000.0
BLK = 1000
GRID = N // BLK

_mesh = plsc.VectorSubcoreMesh(core_axis_name="c", subcore_axis_name="s")


def _stats_body(src3, dst3, nidx_h, ones_h, dout_h, din_h, cnt_h, lastp_h,
                srcp_h, srcl, dstl, wvs, ones_v, nv, lastv_s, srcv_s, vbuf,
                hist_sp, cnt_sp, sem_h, sem_w):
    c = lax.axis_index("c")
    s = lax.axis_index("s")
    for j in range(SL // 16):
        vbuf[pl.ds(16 * j, 16)] = jnp.zeros((16,), jnp.float32)
    pltpu.sync_copy(vbuf, hist_sp.at[pl.ds(s * SL, SL)])
    pltpu.sync_copy(src3.at[s], srcl)
    pltpu.sync_copy(ones_h, ones_v)

    @pl.when(c == 1)
    def _():
        pltpu.sync_copy(vbuf, cnt_sp.at[pl.ds(s * SL, SL)])
        pltpu.sync_copy(dst3.at[s], dstl)
        pltpu.sync_copy(nidx_h, nv)

    plsc.subcore_barrier()

    @pl.when(c == 0)
    def _():
        def bk(k, _):
            pltpu.async_copy(ones_v, hist_sp.at[srcl.at[k]], sem_h, add=True)

            @pl.when(k >= 4)
            def _():
                pltpu.make_async_copy(ones_v, hist_sp.at[srcl.at[0]],
                                      sem_h).wait()
            return 0

        lax.fori_loop(0, NCH, bk, 0)
        for _ in range(4):
            pltpu.make_async_copy(ones_v, hist_sp.at[srcl.at[0]], sem_h).wait()

    @pl.when(c == 1)
    def _():
        nvec = nv[...]
        lanes = lax.iota(jnp.int32, 16)
        base = s * EPT

        def bk(k, carry):
            lastv, srcv = carry
            for j in range(CH // 16):
                s16 = srcl[k, pl.ds(16 * j, 16)]
                d16 = dstl[k, pl.ds(16 * j, 16)]
                m = d16 == nvec
                w = jnp.where(m & (s16 < N - 1), 1.0, 0.0).astype(jnp.float32)
                wvs[k, pl.ds(16 * j, 16)] = w
                eid = base + k * CH + 16 * j + lanes
                upd = m & (eid > lastv)
                lastv = jnp.where(upd, eid, lastv)
                srcv = jnp.where(upd, s16, srcv)
            pltpu.async_copy(ones_v, hist_sp.at[dstl.at[k]], sem_h, add=True)
            pltpu.async_copy(wvs.at[k], cnt_sp.at[srcl.at[k]], sem_w, add=True)

            @pl.when(k >= 4)
            def _():
                pltpu.make_async_copy(ones_v, hist_sp.at[dstl.at[0]],
                                      sem_h).wait()
                pltpu.make_async_copy(wvs.at[0], cnt_sp.at[srcl.at[0]],
                                      sem_w).wait()
            return (lastv, srcv)

        init = (jnp.full((16,), -1, jnp.int32), jnp.full((16,), -1, jnp.int32))
        lastv, srcv = lax.fori_loop(0, NCH, bk, init)
        for _ in range(4):
            pltpu.make_async_copy(ones_v, hist_sp.at[dstl.at[0]], sem_h).wait()
            pltpu.make_async_copy(wvs.at[0], cnt_sp.at[srcl.at[0]],
                                  sem_w).wait()
        lastv_s[...] = lastv
        srcv_s[...] = srcv
        pltpu.sync_copy(lastv_s, lastp_h.at[s])
        pltpu.sync_copy(srcv_s, srcp_h.at[s])

    plsc.subcore_barrier()

    @pl.when(c == 0)
    def _():
        pltpu.sync_copy(hist_sp.at[pl.ds(s * SL, SL)], vbuf)
        pltpu.sync_copy(vbuf, dout_h.at[pl.ds(s * SL, SL)])

    @pl.when(c == 1)
    def _():
        pltpu.sync_copy(hist_sp.at[pl.ds(s * SL, SL)], vbuf)
        pltpu.sync_copy(vbuf, din_h.at[pl.ds(s * SL, SL)])
        pltpu.sync_copy(cnt_sp.at[pl.ds(s * SL, SL)], vbuf)
        pltpu.sync_copy(vbuf, cnt_h.at[pl.ds(s * SL, SL)])


_sc_params = pltpu.CompilerParams(use_tc_tiling_on_sc=False)

_stats_call = functools.partial(
    pl.kernel, _stats_body, mesh=_mesh, compiler_params=_sc_params,
    out_type=(
        jax.ShapeDtypeStruct((NP,), jnp.float32),
        jax.ShapeDtypeStruct((NP,), jnp.float32),
        jax.ShapeDtypeStruct((NP,), jnp.float32),
        jax.ShapeDtypeStruct((NT, 16), jnp.int32),
        jax.ShapeDtypeStruct((NT, 16), jnp.int32),
    ),
    scratch_types=[
        pltpu.VMEM((NCH, CH), jnp.int32),
        pltpu.VMEM((NCH, CH), jnp.int32),
        pltpu.VMEM((NCH, CH), jnp.float32),
        pltpu.VMEM((CH,), jnp.float32),
        pltpu.VMEM((16,), jnp.int32),
        pltpu.VMEM((16,), jnp.int32),
        pltpu.VMEM((16,), jnp.int32),
        pltpu.VMEM((SL,), jnp.float32),
        pltpu.VMEM_SHARED((NP,), jnp.float32),
        pltpu.VMEM_SHARED((NP,), jnp.float32),
        pltpu.SemaphoreType.DMA,
        pltpu.SemaphoreType.DMA,
    ],
)


SLAB = 64
CHS = 125
NCHS = EPT // CHS


def _spmm_body(feat_h, src3, dst3, z640_h, agg_h, srcl, dstl, rows0, rows1,
               rows2, rows3, agg_sp, sem_g, sem_s):
    c = lax.axis_index("c")
    s = lax.axis_index("s")
    rows = (rows0, rows1, rows2, rows3)
    pltpu.sync_copy(src3.at[s], srcl)
    pltpu.sync_copy(dst3.at[s], dstl)

    for p in range(2):
        slab = c * 2 + p
        pltpu.sync_copy(z640_h, agg_sp.at[pl.ds(s * SL, SL)])
        plsc.subcore_barrier()

        fview = feat_h.at[slab]

        def g_start(k, rbuf):
            pltpu.async_copy(fview.at[srcl.at[k]], rbuf, sem_g)

        def g_wait(rbuf):
            pltpu.make_async_copy(fview.at[srcl.at[0]], rbuf, sem_g).wait()

        def s_start(k, rbuf):
            pltpu.async_copy(rbuf, agg_sp.at[dstl.at[k]], sem_s, add=True)

        def s_wait(rbuf):
            pltpu.make_async_copy(rbuf, agg_sp.at[dstl.at[0]], sem_s).wait()

        g_start(0, rows0)
        g_start(1, rows1)

        def body_k(i, _):
            kk = 4 * i
            for b in range(4):
                k = kk + b
                g_wait(rows[b])
                s_start(k, rows[b])

                @pl.when(k >= 2)
                def _():
                    s_wait(rows[(b + 2) % 4])

                @pl.when(k + 2 < NCHS)
                def _():
                    g_start(k + 2, rows[(b + 2) % 4])
            return 0

        lax.fori_loop(0, NCHS // 4, body_k, 0)
        s_wait(rows[(NCHS - 2) % 4])
        s_wait(rows[(NCHS - 1) % 4])
        plsc.subcore_barrier()
        pltpu.sync_copy(agg_sp.at[pl.ds(s * SL, SL)],
                        agg_h.at[slab, pl.ds(s * SL, SL)])
        plsc.subcore_barrier()


_spmm_call = functools.partial(
    pl.kernel, _spmm_body, mesh=_mesh, compiler_params=_sc_params,
    out_type=jax.ShapeDtypeStruct((4, NP, SLAB), jnp.float32),
    scratch_types=[
        pltpu.VMEM((NCHS, CHS), jnp.int32),
        pltpu.VMEM((NCHS, CHS), jnp.int32),
        pltpu.VMEM((CHS, SLAB), jnp.float32),
        pltpu.VMEM((CHS, SLAB), jnp.float32),
        pltpu.VMEM((CHS, SLAB), jnp.float32),
        pltpu.VMEM((CHS, SLAB), jnp.float32),
        pltpu.VMEM_SHARED((NP, SLAB), jnp.float32),
        pltpu.SemaphoreType.DMA,
        pltpu.SemaphoreType.DMA,
    ],
)


def _prep_body(x_ref, dop_ref, dip_ref, feat_ref, doutr_ref, dinr_ref):
    dor = lax.rsqrt(jnp.maximum(dop_ref[...], 1.0))
    dir_ = lax.rsqrt(jnp.maximum(dip_ref[...], 1.0))
    xf = x_ref[...] * dor
    for j in range(4):
        feat_ref[j] = xf[:, SLAB * j:SLAB * (j + 1)]
    doutr_ref[...] = dor
    dinr_ref[...] = dir_


def _prep(x, dout, din):
    return pl.pallas_call(
        _prep_body,
        grid=(GRID,),
        in_specs=[
            pl.BlockSpec((BLK, F), lambda i: (i, 0)),
            pl.BlockSpec((BLK, 1), lambda i: (i, 0)),
            pl.BlockSpec((BLK, 1), lambda i: (i, 0)),
        ],
        out_specs=[
            pl.BlockSpec((4, BLK, SLAB), lambda i: (0, i, 0)),
            pl.BlockSpec((BLK, 1), lambda i: (i, 0)),
            pl.BlockSpec((BLK, 1), lambda i: (i, 0)),
        ],
        out_shape=[
            jax.ShapeDtypeStruct((4, NP, SLAB), jnp.float32),
            jax.ShapeDtypeStruct((N, 1), jnp.float32),
            jax.ShapeDtypeStruct((N, 1), jnp.float32),
        ],
    )(x, dout, din)


def _dense1_body(agg_ref, dinr_ref, doutr_ref, w_ref, b_ref, out_ref):
    z = b_ref[...]
    for j in range(4):
        z = z + jnp.dot(agg_ref[j] * dinr_ref[...], w_ref[j],
                        preferred_element_type=jnp.float32)
    h = jnp.where(z >= 0, z, 0.01 * z) * doutr_ref[...]
    for j in range(4):
        out_ref[j] = h[:, SLAB * j:SLAB * (j + 1)]


def _dense1(agg, dinr, doutr, w, b):
    return pl.pallas_call(
        _dense1_body,
        grid=(GRID,),
        in_specs=[
            pl.BlockSpec((4, BLK, SLAB), lambda i: (0, i, 0)),
            pl.BlockSpec((BLK, 1), lambda i: (i, 0)),
            pl.BlockSpec((BLK, 1), lambda i: (i, 0)),
            pl.BlockSpec((4, SLAB, F), lambda i: (0, 0, 0)),
            pl.BlockSpec((1, F), lambda i: (0, 0)),
        ],
        out_specs=pl.BlockSpec((4, BLK, SLAB), lambda i: (0, i, 0)),
        out_shape=jax.ShapeDtypeStruct((4, NP, SLAB), jnp.float32),
    )(agg, dinr, doutr, w, b)


def _dense2_body(agg_ref, dinr_ref, w_ref, b_ref, wr_ref, out_ref):
    z = b_ref[...]
    for j in range(4):
        z = z + jnp.dot(agg_ref[j] * dinr_ref[...], w_ref[j],
                        preferred_element_type=jnp.float32)
    h = jnp.where(z >= 0, z, 0.01 * z)
    out_ref[...] = jnp.dot(h, wr_ref[...], preferred_element_type=jnp.float32)


def _dense2(agg, dinr, w, b, wr):
    return pl.pallas_call(
        _dense2_body,
        grid=(GRID,),
        in_specs=[
            pl.BlockSpec((4, BLK, SLAB), lambda i: (0, i, 0)),
            pl.BlockSpec((BLK, 1), lambda i: (i, 0)),
            pl.BlockSpec((4, SLAB, F), lambda i: (0, 0, 0)),
            pl.BlockSpec((1, F), lambda i: (0, 0)),
            pl.BlockSpec((F, 2), lambda i: (0, 0)),
        ],
        out_specs=pl.BlockSpec((BLK, 2), lambda i: (i, 0)),
        out_shape=jax.ShapeDtypeStruct((N, 2), jnp.float32),
    )(agg, dinr, w, b, wr)


ROWS = 79
PAD = ROWS * HALF


def _sample_body(s_ref, q_ref, g_ref, cnt_ref, lastp_ref, srcp_ref, nidx_ref,
                 xlast_ref, wrb_ref, br_ref, oh_ref, lp_ref):
    r = lax.broadcasted_iota(jnp.int32, (ROWS, HALF), 0)
    col = lax.broadcasted_iota(jnp.int32, (ROWS, HALF), 1)
    i2 = r * HALF + col
    valid = (i2 >= 8) & (i2 <= N - 2)
    lastm = jnp.max(lastp_ref[...])
    srcl = jnp.sum(jnp.where(lastp_ref[...] == lastm, srcp_ref[...], 0))
    srcl = jnp.where(lastm < 0, -1, srcl)
    cnt = cnt_ref[...] - jnp.where(i2 == srcl, 1.0, 0.0)
    nm = jnp.where(cnt > 0.5, ALPHA, 0.0)
    nidx = jnp.sum(nidx_ref[...])
    csc = (jnp.sum(jnp.where(i2 == nidx, q_ref[...], 0.0))
           + jnp.sum(xlast_ref[...] * wrb_ref[...]) + jnp.sum(br_ref[...]))
    fd = jnp.where(valid, s_ref[...] + csc + nm, -1e30)
    mx = jnp.max(fd)
    e = jnp.where(valid, jnp.exp(fd - mx), 0.0)
    p = e / jnp.sum(e)
    lp = jnp.log(p)
    t = lp + g_ref[...]
    tm = jnp.max(t)
    idxn = jnp.min(jnp.where(t == tm, i2, jnp.int32(2**30)))
    sel = i2 == idxn
    oh_ref[...] = jnp.where(sel, 1.0, 0.0)
    lpv = jnp.sum(jnp.where(sel, jnp.where(valid, lp, 0.0), 0.0))
    lp_ref[...] = jnp.reshape(lpv, (1, 1))


def _sample(s_pad, q_pad, g_node, cnt_pad, lastp, srcp, nidx, xlast, wrb, br2):
    return pl.pallas_call(
        _sample_body,
        out_shape=[
            jax.ShapeDtypeStruct((ROWS, HALF), jnp.float32),
            jax.ShapeDtypeStruct((1, 1), jnp.float32),
        ],
    )(s_pad, q_pad, g_node, cnt_pad, lastp, srcp, nidx, xlast, wrb, br2)


def kernel(x, edge_index, node_index, edge_set, W1, b1, W2, b2, Wr, br):
    f32 = jnp.float32
    src = edge_index[0].astype(jnp.int32)
    dst = edge_index[1].astype(jnp.int32)
    src3 = src.reshape(NT, NCH, CH)
    dst3 = dst.reshape(NT, NCH, CH)
    src3s = src.reshape(NT, NCHS, CHS)
    dst3s = dst.reshape(NT, NCHS, CHS)
    nidx16 = jnp.full((16,), node_index, jnp.int32)
    ones80 = jnp.ones((CH,), f32)

    dout_h, din_h, cnt_h, lastp, srcp = _stats_call()(
        src3, dst3, nidx16, ones80)

    feat, doutr, dinr = _prep(x, dout_h[:N].reshape(N, 1),
                              din_h[:N].reshape(N, 1))
    w1r = W1.reshape(4, SLAB, F)
    w2r = W2.reshape(4, SLAB, F)
    z640 = jnp.zeros((SL, SLAB), f32)
    agg1 = _spmm_call()(feat, src3s, dst3s, z640)
    feat2 = _dense1(agg1, dinr, doutr, w1r, b1.reshape(1, F))
    agg2 = _spmm_call()(feat2, src3s, dst3s, z640)
    wr_ac = jnp.concatenate([Wr[0:F], Wr[2 * F:3 * F]], axis=1)
    sq = _dense2(agg2, dinr, w2r, b2.reshape(1, F), wr_ac)

    zpad = jnp.zeros((PAD - N,), f32)
    s_pad = jnp.concatenate([sq[:, 0], zpad]).reshape(ROWS, HALF)
    q_pad = jnp.concatenate([sq[:, 1], zpad]).reshape(ROWS, HALF)
    cnt_pad = cnt_h[:PAD].reshape(ROWS, HALF)
    g = jax.random.gumbel(jax.random.key(42), (N - 1 - 8,), f32)
    g_node = jnp.concatenate(
        [jnp.zeros((8,), f32), g, jnp.zeros((PAD - (N - 1),), f32)]
    ).reshape(ROWS, HALF)
    nidx11 = jnp.asarray(node_index, jnp.int32).reshape(1, 1)
    xlast = x[-1].reshape(2, HALF)
    wrb = Wr[F:2 * F, 0].reshape(2, HALF)
    br2 = br.reshape(1, 1)

    oh, lp = _sample(s_pad, q_pad, g_node, cnt_pad, lastp, srcp, nidx11,
                     xlast, wrb, br2)
    sample_full = oh.reshape(PAD)[:N - 1]
    log_prob = lp.reshape(())
    return (sample_full, log_prob)


# speedup vs baseline: 7.4251x; 1.1583x over previous
import functools

import jax
import jax.numpy as jnp
from jax import lax
from jax.experimental import pallas as pl
from jax.experimental.pallas import tpu as pltpu
from jax.experimental.pallas import tpu_sc as plsc

N = 10000
NP = 10240
E = 160000
F = 256
HALF = 128
NT = 16
EPT = E // NT
CH = 80
NCH = EPT // CH
SL = NP // NT
ALPHA = 1000000.0
BLK = 1000
GRID = N // BLK

_mesh = plsc.VectorSubcoreMesh(core_axis_name="c", subcore_axis_name="s")


def _stats_body(src3, dst3, nidx_h, ones_h, dout_h, din_h, cnt_h, lastp_h,
                srcp_h, srcl, dstl, wvs, ones_v, nv, lastv_s, srcv_s, vbuf,
                hist_sp, cnt_sp, sem_h, sem_w):
    c = lax.axis_index("c")
    s = lax.axis_index("s")
    for j in range(SL // 16):
        vbuf[pl.ds(16 * j, 16)] = jnp.zeros((16,), jnp.float32)
    pltpu.sync_copy(vbuf, hist_sp.at[pl.ds(s * SL, SL)])
    pltpu.sync_copy(src3.at[s], srcl)
    pltpu.sync_copy(ones_h, ones_v)

    @pl.when(c == 1)
    def _():
        pltpu.sync_copy(vbuf, cnt_sp.at[pl.ds(s * SL, SL)])
        pltpu.sync_copy(dst3.at[s], dstl)
        pltpu.sync_copy(nidx_h, nv)

    plsc.subcore_barrier()

    @pl.when(c == 0)
    def _():
        def bk(k, _):
            pltpu.async_copy(ones_v, hist_sp.at[srcl.at[k]], sem_h, add=True)

            @pl.when(k >= 4)
            def _():
                pltpu.make_async_copy(ones_v, hist_sp.at[srcl.at[0]],
                                      sem_h).wait()
            return 0

        lax.fori_loop(0, NCH, bk, 0)
        for _ in range(4):
            pltpu.make_async_copy(ones_v, hist_sp.at[srcl.at[0]], sem_h).wait()

    @pl.when(c == 1)
    def _():
        nvec = nv[...]
        lanes = lax.iota(jnp.int32, 16)
        base = s * EPT

        def bk(k, carry):
            lastv, srcv = carry
            for j in range(CH // 16):
                s16 = srcl[k, pl.ds(16 * j, 16)]
                d16 = dstl[k, pl.ds(16 * j, 16)]
                m = d16 == nvec
                w = jnp.where(m & (s16 < N - 1), 1.0, 0.0).astype(jnp.float32)
                wvs[k, pl.ds(16 * j, 16)] = w
                eid = base + k * CH + 16 * j + lanes
                upd = m & (eid > lastv)
                lastv = jnp.where(upd, eid, lastv)
                srcv = jnp.where(upd, s16, srcv)
            pltpu.async_copy(ones_v, hist_sp.at[dstl.at[k]], sem_h, add=True)
            pltpu.async_copy(wvs.at[k], cnt_sp.at[srcl.at[k]], sem_w, add=True)

            @pl.when(k >= 4)
            def _():
                pltpu.make_async_copy(ones_v, hist_sp.at[dstl.at[0]],
                                      sem_h).wait()
                pltpu.make_async_copy(wvs.at[0], cnt_sp.at[srcl.at[0]],
                                      sem_w).wait()
            return (lastv, srcv)

        init = (jnp.full((16,), -1, jnp.int32), jnp.full((16,), -1, jnp.int32))
        lastv, srcv = lax.fori_loop(0, NCH, bk, init)
        for _ in range(4):
            pltpu.make_async_copy(ones_v, hist_sp.at[dstl.at[0]], sem_h).wait()
            pltpu.make_async_copy(wvs.at[0], cnt_sp.at[srcl.at[0]],
                                  sem_w).wait()
        lastv_s[...] = lastv
        srcv_s[...] = srcv
        pltpu.sync_copy(lastv_s, lastp_h.at[s])
        pltpu.sync_copy(srcv_s, srcp_h.at[s])

    plsc.subcore_barrier()

    @pl.when(c == 0)
    def _():
        pltpu.sync_copy(hist_sp.at[pl.ds(s * SL, SL)], vbuf)
        pltpu.sync_copy(vbuf, dout_h.at[pl.ds(s * SL, SL)])

    @pl.when(c == 1)
    def _():
        pltpu.sync_copy(hist_sp.at[pl.ds(s * SL, SL)], vbuf)
        pltpu.sync_copy(vbuf, din_h.at[pl.ds(s * SL, SL)])
        pltpu.sync_copy(cnt_sp.at[pl.ds(s * SL, SL)], vbuf)
        pltpu.sync_copy(vbuf, cnt_h.at[pl.ds(s * SL, SL)])


_sc_params = pltpu.CompilerParams(use_tc_tiling_on_sc=False)

_stats_call = functools.partial(
    pl.kernel, _stats_body, mesh=_mesh, compiler_params=_sc_params,
    out_type=(
        jax.ShapeDtypeStruct((NP,), jnp.float32),
        jax.ShapeDtypeStruct((NP,), jnp.float32),
        jax.ShapeDtypeStruct((NP,), jnp.float32),
        jax.ShapeDtypeStruct((NT, 16), jnp.int32),
        jax.ShapeDtypeStruct((NT, 16), jnp.int32),
    ),
    scratch_types=[
        pltpu.VMEM((NCH, CH), jnp.int32),
        pltpu.VMEM((NCH, CH), jnp.int32),
        pltpu.VMEM((NCH, CH), jnp.float32),
        pltpu.VMEM((CH,), jnp.float32),
        pltpu.VMEM((16,), jnp.int32),
        pltpu.VMEM((16,), jnp.int32),
        pltpu.VMEM((16,), jnp.int32),
        pltpu.VMEM((SL,), jnp.float32),
        pltpu.VMEM_SHARED((NP,), jnp.float32),
        pltpu.VMEM_SHARED((NP,), jnp.float32),
        pltpu.SemaphoreType.DMA,
        pltpu.SemaphoreType.DMA,
    ],
)


SLAB = 64
CHS = 125
NCHS = EPT // CHS


NBUF = 8


def _spmm_body(feat_h, src3, dst3, z640_h, agg_h, srcl, dstl, *rest):
    rows = rest[:NBUF]
    agg_sp, sem_g, sem_s = rest[NBUF:]
    c = lax.axis_index("c")
    s = lax.axis_index("s")
    pltpu.sync_copy(src3.at[s], srcl)
    pltpu.sync_copy(dst3.at[s], dstl)

    for p in range(2):
        slab = c * 2 + p
        pltpu.sync_copy(z640_h, agg_sp.at[pl.ds(s * SL, SL)])
        plsc.subcore_barrier()

        fview = feat_h.at[slab]

        def g_start(k, rbuf):
            pltpu.async_copy(fview.at[srcl.at[k]], rbuf, sem_g)

        def g_wait(rbuf):
            pltpu.make_async_copy(fview.at[srcl.at[0]], rbuf, sem_g).wait()

        def s_start(k, rbuf):
            pltpu.async_copy(rbuf, agg_sp.at[dstl.at[k]], sem_s, add=True)

        def s_wait(rbuf):
            pltpu.make_async_copy(rbuf, agg_sp.at[dstl.at[0]], sem_s).wait()

        for b in range(4):
            g_start(b, rows[b])

        def body_k(i, _):
            kk = NBUF * i
            for b in range(NBUF):
                k = kk + b
                g_wait(rows[b])
                s_start(k, rows[b])

                @pl.when(k >= 4)
                def _():
                    s_wait(rows[(b + 4) % NBUF])

                @pl.when(k + 4 < NCHS)
                def _():
                    g_start(k + 4, rows[(b + 4) % NBUF])
            return 0

        lax.fori_loop(0, NCHS // NBUF, body_k, 0)
        for k in range(NCHS - 4, NCHS):
            s_wait(rows[k % NBUF])
        plsc.subcore_barrier()
        pltpu.sync_copy(agg_sp.at[pl.ds(s * SL, SL)],
                        agg_h.at[slab, pl.ds(s * SL, SL)])
        plsc.subcore_barrier()


_spmm_call = functools.partial(
    pl.kernel, _spmm_body, mesh=_mesh, compiler_params=_sc_params,
    out_type=jax.ShapeDtypeStruct((4, NP, SLAB), jnp.float32),
    scratch_types=[
        pltpu.VMEM((NCHS, CHS), jnp.int32),
        pltpu.VMEM((NCHS, CHS), jnp.int32),
    ] + [pltpu.VMEM((CHS, SLAB), jnp.float32) for _ in range(NBUF)] + [
        pltpu.VMEM_SHARED((NP, SLAB), jnp.float32),
        pltpu.SemaphoreType.DMA,
        pltpu.SemaphoreType.DMA,
    ],
)


BLK2 = 2048
GRID2 = NP // BLK2


def _prep_body(x_ref, dop_ref, dip_ref, feat_ref, doutr_ref, dinr_ref):
    dor = lax.rsqrt(jnp.maximum(dop_ref[...], 1.0))
    dir_ = lax.rsqrt(jnp.maximum(dip_ref[...], 1.0))
    x3 = x_ref[...].reshape(16, HALF, F)
    xf = x3 * dor[:, :, None]
    for j in range(4):
        feat_ref[j] = xf[:, :, SLAB * j:SLAB * (j + 1)].reshape(BLK2, SLAB)
    doutr_ref[...] = dor
    dinr_ref[...] = dir_


def _prep(x, dout, din):
    return pl.pallas_call(
        _prep_body,
        grid=(GRID2,),
        in_specs=[
            pl.BlockSpec((BLK2, F), lambda i: (i, 0)),
            pl.BlockSpec((16, HALF), lambda i: (i, 0)),
            pl.BlockSpec((16, HALF), lambda i: (i, 0)),
        ],
        out_specs=[
            pl.BlockSpec((4, BLK2, SLAB), lambda i: (0, i, 0)),
            pl.BlockSpec((16, HALF), lambda i: (i, 0)),
            pl.BlockSpec((16, HALF), lambda i: (i, 0)),
        ],
        out_shape=[
            jax.ShapeDtypeStruct((4, NP, SLAB), jnp.float32),
            jax.ShapeDtypeStruct((80, HALF), jnp.float32),
            jax.ShapeDtypeStruct((80, HALF), jnp.float32),
        ],
    )(x, dout, din)


def _dense1_body(agg_ref, dinr_ref, doutr_ref, w_ref, b_ref, out_ref):
    dinc = dinr_ref[...][:, :, None]
    z = b_ref[...]
    for j in range(4):
        a = (agg_ref[j].reshape(16, HALF, SLAB) * dinc).reshape(BLK2, SLAB)
        z = z + jnp.dot(a, w_ref[j], preferred_element_type=jnp.float32)
    h = jnp.where(z >= 0, z, 0.01 * z).reshape(16, HALF, F)
    h = h * doutr_ref[...][:, :, None]
    for j in range(4):
        out_ref[j] = h[:, :, SLAB * j:SLAB * (j + 1)].reshape(BLK2, SLAB)


def _dense1(agg, dinr, doutr, w, b):
    return pl.pallas_call(
        _dense1_body,
        grid=(GRID2,),
        in_specs=[
            pl.BlockSpec((4, BLK2, SLAB), lambda i: (0, i, 0)),
            pl.BlockSpec((16, HALF), lambda i: (i, 0)),
            pl.BlockSpec((16, HALF), lambda i: (i, 0)),
            pl.BlockSpec((4, SLAB, F), lambda i: (0, 0, 0)),
            pl.BlockSpec((1, F), lambda i: (0, 0)),
        ],
        out_specs=pl.BlockSpec((4, BLK2, SLAB), lambda i: (0, i, 0)),
        out_shape=jax.ShapeDtypeStruct((4, NP, SLAB), jnp.float32),
    )(agg, dinr, doutr, w, b)


def _dense2_body(agg_ref, dinr_ref, w_ref, b_ref, wr_ref, s_ref, q_ref):
    dinc = dinr_ref[...][:, :, None]
    z = b_ref[...]
    for j in range(4):
        a = (agg_ref[j].reshape(16, HALF, SLAB) * dinc).reshape(BLK2, SLAB)
        z = z + jnp.dot(a, w_ref[j], preferred_element_type=jnp.float32)
    h = jnp.where(z >= 0, z, 0.01 * z)
    sq = jnp.dot(h, wr_ref[...], preferred_element_type=jnp.float32)
    s_ref[...] = sq[:, 0].reshape(16, HALF)
    q_ref[...] = sq[:, 1].reshape(16, HALF)


def _dense2(agg, dinr, w, b, wr):
    return pl.pallas_call(
        _dense2_body,
        grid=(GRID2,),
        in_specs=[
            pl.BlockSpec((4, BLK2, SLAB), lambda i: (0, i, 0)),
            pl.BlockSpec((16, HALF), lambda i: (i, 0)),
            pl.BlockSpec((4, SLAB, F), lambda i: (0, 0, 0)),
            pl.BlockSpec((1, F), lambda i: (0, 0)),
            pl.BlockSpec((F, 2), lambda i: (0, 0)),
        ],
        out_specs=[
            pl.BlockSpec((16, HALF), lambda i: (i, 0)),
            pl.BlockSpec((16, HALF), lambda i: (i, 0)),
        ],
        out_shape=[
            jax.ShapeDtypeStruct((80, HALF), jnp.float32),
            jax.ShapeDtypeStruct((80, HALF), jnp.float32),
        ],
    )(agg, dinr, w, b, wr)


ROWS = 80
PAD = ROWS * HALF


def _sample_body(s_ref, q_ref, g_ref, cnt_ref, lastp_ref, srcp_ref, nidx_ref,
                 xlast_ref, wrb_ref, br_ref, oh_ref, lp_ref):
    r = lax.broadcasted_iota(jnp.int32, (ROWS, HALF), 0)
    col = lax.broadcasted_iota(jnp.int32, (ROWS, HALF), 1)
    i2 = r * HALF + col
    valid = (i2 >= 8) & (i2 <= N - 2)
    lastm = jnp.max(lastp_ref[...])
    srcl = jnp.sum(jnp.where(lastp_ref[...] == lastm, srcp_ref[...], 0))
    srcl = jnp.where(lastm < 0, -1, srcl)
    cnt = cnt_ref[...] - jnp.where(i2 == srcl, 1.0, 0.0)
    nm = jnp.where(cnt > 0.5, ALPHA, 0.0)
    nidx = jnp.sum(nidx_ref[...])
    csc = (jnp.sum(jnp.where(i2 == nidx, q_ref[...], 0.0))
           + jnp.sum(xlast_ref[...] * wrb_ref[...]) + jnp.sum(br_ref[...]))
    fd = jnp.where(valid, s_ref[...] + csc + nm, -1e30)
    mx = jnp.max(fd)
    e = jnp.where(valid, jnp.exp(fd - mx), 0.0)
    p = e / jnp.sum(e)
    lp = jnp.log(p)
    t = lp + g_ref[...]
    tm = jnp.max(t)
    idxn = jnp.min(jnp.where(t == tm, i2, jnp.int32(2**30)))
    sel = i2 == idxn
    oh_ref[...] = jnp.where(sel, 1.0, 0.0)
    lpv = jnp.sum(jnp.where(sel, jnp.where(valid, lp, 0.0), 0.0))
    lp_ref[...] = jnp.reshape(lpv, (1, 1))


def _sample(s_pad, q_pad, g_node, cnt_pad, lastp, srcp, nidx, xlast, wrb, br2):
    return pl.pallas_call(
        _sample_body,
        out_shape=[
            jax.ShapeDtypeStruct((ROWS, HALF), jnp.float32),
            jax.ShapeDtypeStruct((1, 1), jnp.float32),
        ],
    )(s_pad, q_pad, g_node, cnt_pad, lastp, srcp, nidx, xlast, wrb, br2)


def kernel(x, edge_index, node_index, edge_set, W1, b1, W2, b2, Wr, br):
    f32 = jnp.float32
    src = edge_index[0].astype(jnp.int32)
    dst = edge_index[1].astype(jnp.int32)
    src3 = src.reshape(NT, NCH, CH)
    dst3 = dst.reshape(NT, NCH, CH)
    src3s = src.reshape(NT, NCHS, CHS)
    dst3s = dst.reshape(NT, NCHS, CHS)
    nidx16 = jnp.full((16,), node_index, jnp.int32)
    ones80 = jnp.ones((CH,), f32)

    dout_h, din_h, cnt_h, lastp, srcp = _stats_call()(
        src3, dst3, nidx16, ones80)

    feat, doutr, dinr = _prep(x, dout_h.reshape(ROWS, HALF),
                              din_h.reshape(ROWS, HALF))
    w1r = W1.reshape(4, SLAB, F)
    w2r = W2.reshape(4, SLAB, F)
    z640 = jnp.zeros((SL, SLAB), f32)
    agg1 = _spmm_call()(feat, src3s, dst3s, z640)
    feat2 = _dense1(agg1, dinr, doutr, w1r, b1.reshape(1, F))
    agg2 = _spmm_call()(feat2, src3s, dst3s, z640)
    wr_ac = jnp.concatenate([Wr[0:F], Wr[2 * F:3 * F]], axis=1)
    s_pad, q_pad = _dense2(agg2, dinr, w2r, b2.reshape(1, F), wr_ac)
    cnt_pad = cnt_h.reshape(ROWS, HALF)
    g = jax.random.gumbel(jax.random.key(42), (N - 1 - 8,), f32)
    g_node = jnp.concatenate(
        [jnp.zeros((8,), f32), g, jnp.zeros((PAD - (N - 1),), f32)]
    ).reshape(ROWS, HALF)
    nidx11 = jnp.asarray(node_index, jnp.int32).reshape(1, 1)
    xlast = x[-1].reshape(2, HALF)
    wrb = Wr[F:2 * F, 0].reshape(2, HALF)
    br2 = br.reshape(1, 1)

    oh, lp = _sample(s_pad, q_pad, g_node, cnt_pad, lastp, srcp, nidx11,
                     xlast, wrb, br2)
    sample_full = oh.reshape(PAD)[:N - 1]
    log_prob = lp.reshape(())
    return (sample_full, log_prob)


# speedup vs baseline: 7.4293x; 1.0006x over previous
import functools

import jax
import jax.numpy as jnp
from jax import lax
from jax.experimental import pallas as pl
from jax.experimental.pallas import tpu as pltpu
from jax.experimental.pallas import tpu_sc as plsc

N = 10000
NP = 10240
E = 160000
F = 256
HALF = 128
NT = 16
EPT = E // NT
CH = 80
NCH = EPT // CH
SL = NP // NT
ALPHA = 1000000.0
BLK = 1000
GRID = N // BLK

_mesh = plsc.VectorSubcoreMesh(core_axis_name="c", subcore_axis_name="s")


def _stats_body(src3, dst3, nidx_h, ones_h, dout_h, din_h, cnt_h, lastp_h,
                srcp_h, srcl, dstl, wvs, ones_v, nv, lastv_s, srcv_s, vbuf,
                hist_sp, cnt_sp, sem_h, sem_w):
    c = lax.axis_index("c")
    s = lax.axis_index("s")
    for j in range(SL // 16):
        vbuf[pl.ds(16 * j, 16)] = jnp.zeros((16,), jnp.float32)
    pltpu.sync_copy(vbuf, hist_sp.at[pl.ds(s * SL, SL)])
    pltpu.sync_copy(src3.at[s], srcl)
    pltpu.sync_copy(ones_h, ones_v)

    @pl.when(c == 1)
    def _():
        pltpu.sync_copy(vbuf, cnt_sp.at[pl.ds(s * SL, SL)])
        pltpu.sync_copy(dst3.at[s], dstl)
        pltpu.sync_copy(nidx_h, nv)

    plsc.subcore_barrier()

    @pl.when(c == 0)
    def _():
        def bk(k, _):
            pltpu.async_copy(ones_v, hist_sp.at[srcl.at[k]], sem_h, add=True)

            @pl.when(k >= 4)
            def _():
                pltpu.make_async_copy(ones_v, hist_sp.at[srcl.at[0]],
                                      sem_h).wait()
            return 0

        lax.fori_loop(0, NCH, bk, 0)
        for _ in range(4):
            pltpu.make_async_copy(ones_v, hist_sp.at[srcl.at[0]], sem_h).wait()

    @pl.when(c == 1)
    def _():
        nvec = nv[...]
        lanes = lax.iota(jnp.int32, 16)
        base = s * EPT

        def bk(k, carry):
            lastv, srcv = carry
            for j in range(CH // 16):
                s16 = srcl[k, pl.ds(16 * j, 16)]
                d16 = dstl[k, pl.ds(16 * j, 16)]
                m = d16 == nvec
                w = jnp.where(m & (s16 < N - 1), 1.0, 0.0).astype(jnp.float32)
                wvs[k, pl.ds(16 * j, 16)] = w
                eid = base + k * CH + 16 * j + lanes
                upd = m & (eid > lastv)
                lastv = jnp.where(upd, eid, lastv)
                srcv = jnp.where(upd, s16, srcv)
            pltpu.async_copy(ones_v, hist_sp.at[dstl.at[k]], sem_h, add=True)
            pltpu.async_copy(wvs.at[k], cnt_sp.at[srcl.at[k]], sem_w, add=True)

            @pl.when(k >= 4)
            def _():
                pltpu.make_async_copy(ones_v, hist_sp.at[dstl.at[0]],
                                      sem_h).wait()
                pltpu.make_async_copy(wvs.at[0], cnt_sp.at[srcl.at[0]],
                                      sem_w).wait()
            return (lastv, srcv)

        init = (jnp.full((16,), -1, jnp.int32), jnp.full((16,), -1, jnp.int32))
        lastv, srcv = lax.fori_loop(0, NCH, bk, init)
        for _ in range(4):
            pltpu.make_async_copy(ones_v, hist_sp.at[dstl.at[0]], sem_h).wait()
            pltpu.make_async_copy(wvs.at[0], cnt_sp.at[srcl.at[0]],
                                  sem_w).wait()
        lastv_s[...] = lastv
        srcv_s[...] = srcv
        pltpu.sync_copy(lastv_s, lastp_h.at[s])
        pltpu.sync_copy(srcv_s, srcp_h.at[s])

    plsc.subcore_barrier()

    @pl.when(c == 0)
    def _():
        pltpu.sync_copy(hist_sp.at[pl.ds(s * SL, SL)], vbuf)
        pltpu.sync_copy(vbuf, dout_h.at[pl.ds(s * SL, SL)])

    @pl.when(c == 1)
    def _():
        pltpu.sync_copy(hist_sp.at[pl.ds(s * SL, SL)], vbuf)
        pltpu.sync_copy(vbuf, din_h.at[pl.ds(s * SL, SL)])
        pltpu.sync_copy(cnt_sp.at[pl.ds(s * SL, SL)], vbuf)
        pltpu.sync_copy(vbuf, cnt_h.at[pl.ds(s * SL, SL)])


_sc_params = pltpu.CompilerParams(use_tc_tiling_on_sc=False)

_stats_call = functools.partial(
    pl.kernel, _stats_body, mesh=_mesh, compiler_params=_sc_params,
    out_type=(
        jax.ShapeDtypeStruct((NP,), jnp.float32),
        jax.ShapeDtypeStruct((NP,), jnp.float32),
        jax.ShapeDtypeStruct((NP,), jnp.float32),
        jax.ShapeDtypeStruct((NT, 16), jnp.int32),
        jax.ShapeDtypeStruct((NT, 16), jnp.int32),
    ),
    scratch_types=[
        pltpu.VMEM((NCH, CH), jnp.int32),
        pltpu.VMEM((NCH, CH), jnp.int32),
        pltpu.VMEM((NCH, CH), jnp.float32),
        pltpu.VMEM((CH,), jnp.float32),
        pltpu.VMEM((16,), jnp.int32),
        pltpu.VMEM((16,), jnp.int32),
        pltpu.VMEM((16,), jnp.int32),
        pltpu.VMEM((SL,), jnp.float32),
        pltpu.VMEM_SHARED((NP,), jnp.float32),
        pltpu.VMEM_SHARED((NP,), jnp.float32),
        pltpu.SemaphoreType.DMA,
        pltpu.SemaphoreType.DMA,
    ],
)


SLAB = 64
CHS = 125
NCHS = EPT // CHS


NBUF = 8


def _spmm_body(feat_h, src3, dst3, z640_h, agg_h, srcl, dstl, *rest):
    rows = rest[:NBUF]
    agg_sp, sem_g, sem_s = rest[NBUF:]
    c = lax.axis_index("c")
    s = lax.axis_index("s")
    pltpu.sync_copy(src3.at[s], srcl)
    pltpu.sync_copy(dst3.at[s], dstl)

    def make_ops(slab):
        fview = feat_h.at[slab]

        def g_start(k, rbuf):
            pltpu.async_copy(fview.at[srcl.at[k]], rbuf, sem_g)

        def g_wait(rbuf):
            pltpu.make_async_copy(fview.at[srcl.at[0]], rbuf, sem_g).wait()

        def s_start(k, rbuf):
            pltpu.async_copy(rbuf, agg_sp.at[dstl.at[k]], sem_s, add=True)

        def s_wait(rbuf):
            pltpu.make_async_copy(rbuf, agg_sp.at[dstl.at[0]], sem_s).wait()

        return g_start, g_wait, s_start, s_wait

    for p in range(2):
        slab = c * 2 + p
        g_start, g_wait, s_start, s_wait = make_ops(slab)
        pltpu.sync_copy(z640_h, agg_sp.at[pl.ds(s * SL, SL)])
        plsc.subcore_barrier()
        for b in range(4):
            g_start(b, rows[b])

        def body_k(i, _):
            kk = NBUF * i
            for b in range(NBUF):
                k = kk + b
                g_wait(rows[b])
                s_start(k, rows[b])

                @pl.when(k >= 4)
                def _():
                    s_wait(rows[(b + 4) % NBUF])

                @pl.when(k + 4 < NCHS)
                def _():
                    g_start(k + 4, rows[(b + 4) % NBUF])
            return 0

        lax.fori_loop(0, NCHS // NBUF, body_k, 0)
        for k in range(NCHS - 4, NCHS):
            s_wait(rows[k % NBUF])
        plsc.subcore_barrier()
        pltpu.sync_copy(agg_sp.at[pl.ds(s * SL, SL)],
                        agg_h.at[slab, pl.ds(s * SL, SL)])
        plsc.subcore_barrier()


_spmm_call = functools.partial(
    pl.kernel, _spmm_body, mesh=_mesh, compiler_params=_sc_params,
    out_type=jax.ShapeDtypeStruct((4, NP, SLAB), jnp.float32),
    scratch_types=[
        pltpu.VMEM((NCHS, CHS), jnp.int32),
        pltpu.VMEM((NCHS, CHS), jnp.int32),
    ] + [pltpu.VMEM((CHS, SLAB), jnp.float32) for _ in range(NBUF)] + [
        pltpu.VMEM_SHARED((NP, SLAB), jnp.float32),
        pltpu.SemaphoreType.DMA,
        pltpu.SemaphoreType.DMA,
    ],
)


BLK2 = 2048
GRID2 = NP // BLK2


def _prep_body(x_ref, dop_ref, dip_ref, feat_ref, doutr_ref, dinr_ref):
    dor = lax.rsqrt(jnp.maximum(dop_ref[...], 1.0))
    dir_ = lax.rsqrt(jnp.maximum(dip_ref[...], 1.0))
    x3 = x_ref[...].reshape(16, HALF, F)
    xf = x3 * dor[:, :, None]
    for j in range(4):
        feat_ref[j] = xf[:, :, SLAB * j:SLAB * (j + 1)].reshape(BLK2, SLAB)
    doutr_ref[...] = dor
    dinr_ref[...] = dir_


def _prep(x, dout, din):
    return pl.pallas_call(
        _prep_body,
        grid=(GRID2,),
        in_specs=[
            pl.BlockSpec((BLK2, F), lambda i: (i, 0)),
            pl.BlockSpec((16, HALF), lambda i: (i, 0)),
            pl.BlockSpec((16, HALF), lambda i: (i, 0)),
        ],
        out_specs=[
            pl.BlockSpec((4, BLK2, SLAB), lambda i: (0, i, 0)),
            pl.BlockSpec((16, HALF), lambda i: (i, 0)),
            pl.BlockSpec((16, HALF), lambda i: (i, 0)),
        ],
        out_shape=[
            jax.ShapeDtypeStruct((4, NP, SLAB), jnp.float32),
            jax.ShapeDtypeStruct((80, HALF), jnp.float32),
            jax.ShapeDtypeStruct((80, HALF), jnp.float32),
        ],
    )(x, dout, din)


def _dense1_body(agg_ref, dinr_ref, doutr_ref, w_ref, b_ref, out_ref):
    dinc = dinr_ref[...][:, :, None]
    z = b_ref[...]
    for j in range(4):
        a = (agg_ref[j].reshape(16, HALF, SLAB) * dinc).reshape(BLK2, SLAB)
        z = z + jnp.dot(a, w_ref[j], preferred_element_type=jnp.float32)
    h = jnp.where(z >= 0, z, 0.01 * z).reshape(16, HALF, F)
    h = h * doutr_ref[...][:, :, None]
    for j in range(4):
        out_ref[j] = h[:, :, SLAB * j:SLAB * (j + 1)].reshape(BLK2, SLAB)


def _dense1(agg, dinr, doutr, w, b):
    return pl.pallas_call(
        _dense1_body,
        grid=(GRID2,),
        in_specs=[
            pl.BlockSpec((4, BLK2, SLAB), lambda i: (0, i, 0)),
            pl.BlockSpec((16, HALF), lambda i: (i, 0)),
            pl.BlockSpec((16, HALF), lambda i: (i, 0)),
            pl.BlockSpec((4, SLAB, F), lambda i: (0, 0, 0)),
            pl.BlockSpec((1, F), lambda i: (0, 0)),
        ],
        out_specs=pl.BlockSpec((4, BLK2, SLAB), lambda i: (0, i, 0)),
        out_shape=jax.ShapeDtypeStruct((4, NP, SLAB), jnp.float32),
    )(agg, dinr, doutr, w, b)


ROWS = 80
PAD = ROWS * HALF


def _score_body(agg_ref, dinr_ref, w_ref, b_ref, wr_ref, g_ref, cnt_ref,
                lastp_ref, srcp_ref, nidx_ref, xlast_ref, wrb_ref, br_ref,
                oh_ref, lp_ref, s_sc, q_sc):
    i = pl.program_id(0)
    dinc = dinr_ref[...][:, :, None]
    z = b_ref[...]
    for j in range(4):
        a = (agg_ref[j].reshape(16, HALF, SLAB) * dinc).reshape(BLK2, SLAB)
        z = z + jnp.dot(a, w_ref[j], preferred_element_type=jnp.float32)
    h = jnp.where(z >= 0, z, 0.01 * z)
    sq = jnp.dot(h, wr_ref[...], preferred_element_type=jnp.float32)
    s_sc[pl.ds(16 * i, 16), :] = sq[:, 0].reshape(16, HALF)
    q_sc[pl.ds(16 * i, 16), :] = sq[:, 1].reshape(16, HALF)

    @pl.when(i == GRID2 - 1)
    def _():
        r = lax.broadcasted_iota(jnp.int32, (ROWS, HALF), 0)
        col = lax.broadcasted_iota(jnp.int32, (ROWS, HALF), 1)
        i2 = r * HALF + col
        valid = (i2 >= 8) & (i2 <= N - 2)
        lastm = jnp.max(lastp_ref[...])
        srcl = jnp.sum(jnp.where(lastp_ref[...] == lastm, srcp_ref[...], 0))
        srcl = jnp.where(lastm < 0, -1, srcl)
        cnt = cnt_ref[...] - jnp.where(i2 == srcl, 1.0, 0.0)
        nm = jnp.where(cnt > 0.5, ALPHA, 0.0)
        nidx = jnp.sum(nidx_ref[...])
        csc = (jnp.sum(jnp.where(i2 == nidx, q_sc[...], 0.0))
               + jnp.sum(xlast_ref[...] * wrb_ref[...]) + jnp.sum(br_ref[...]))
        fd = jnp.where(valid, s_sc[...] + csc + nm, -1e30)
        mx = jnp.max(fd)
        e = jnp.where(valid, jnp.exp(fd - mx), 0.0)
        p = e / jnp.sum(e)
        lp = jnp.log(p)
        t = lp + g_ref[...]
        tm = jnp.max(t)
        idxn = jnp.min(jnp.where(t == tm, i2, jnp.int32(2**30)))
        sel = i2 == idxn
        oh_ref[...] = jnp.where(sel, 1.0, 0.0)
        lpv = jnp.sum(jnp.where(sel, jnp.where(valid, lp, 0.0), 0.0))
        lp_ref[...] = jnp.reshape(lpv, (1, 1))


def _score(agg, dinr, w, b, wr, g_node, cnt, lastp, srcp, nidx, xlast, wrb,
           br2):
    return pl.pallas_call(
        _score_body,
        grid=(GRID2,),
        in_specs=[
            pl.BlockSpec((4, BLK2, SLAB), lambda i: (0, i, 0)),
            pl.BlockSpec((16, HALF), lambda i: (i, 0)),
            pl.BlockSpec((4, SLAB, F), lambda i: (0, 0, 0)),
            pl.BlockSpec((1, F), lambda i: (0, 0)),
            pl.BlockSpec((F, 2), lambda i: (0, 0)),
            pl.BlockSpec((ROWS, HALF), lambda i: (0, 0)),
            pl.BlockSpec((ROWS, HALF), lambda i: (0, 0)),
            pl.BlockSpec((NT, 16), lambda i: (0, 0)),
            pl.BlockSpec((NT, 16), lambda i: (0, 0)),
            pl.BlockSpec((1, 1), lambda i: (0, 0)),
            pl.BlockSpec((2, HALF), lambda i: (0, 0)),
            pl.BlockSpec((2, HALF), lambda i: (0, 0)),
            pl.BlockSpec((1, 1), lambda i: (0, 0)),
        ],
        out_specs=[
            pl.BlockSpec((ROWS, HALF), lambda i: (0, 0)),
            pl.BlockSpec((1, 1), lambda i: (0, 0)),
        ],
        out_shape=[
            jax.ShapeDtypeStruct((ROWS, HALF), jnp.float32),
            jax.ShapeDtypeStruct((1, 1), jnp.float32),
        ],
        scratch_shapes=[
            pltpu.VMEM((ROWS, HALF), jnp.float32),
            pltpu.VMEM((ROWS, HALF), jnp.float32),
        ],
    )(agg, dinr, w, b, wr, g_node, cnt, lastp, srcp, nidx, xlast, wrb, br2)


def kernel(x, edge_index, node_index, edge_set, W1, b1, W2, b2, Wr, br):
    f32 = jnp.float32
    src = edge_index[0].astype(jnp.int32)
    dst = edge_index[1].astype(jnp.int32)
    src3 = src.reshape(NT, NCH, CH)
    dst3 = dst.reshape(NT, NCH, CH)
    src3s = src.reshape(NT, NCHS, CHS)
    dst3s = dst.reshape(NT, NCHS, CHS)
    nidx16 = jnp.full((16,), node_index, jnp.int32)
    ones80 = jnp.ones((CH,), f32)

    dout_h, din_h, cnt_h, lastp, srcp = _stats_call()(
        src3, dst3, nidx16, ones80)

    feat, doutr, dinr = _prep(x, dout_h.reshape(ROWS, HALF),
                              din_h.reshape(ROWS, HALF))
    w1r = W1.reshape(4, SLAB, F)
    w2r = W2.reshape(4, SLAB, F)
    z640 = jnp.zeros((SL, SLAB), f32)
    agg1 = _spmm_call()(feat, src3s, dst3s, z640)
    feat2 = _dense1(agg1, dinr, doutr, w1r, b1.reshape(1, F))
    agg2 = _spmm_call()(feat2, src3s, dst3s, z640)
    wr_ac = jnp.concatenate([Wr[0:F], Wr[2 * F:3 * F]], axis=1)
    cnt_pad = cnt_h.reshape(ROWS, HALF)
    g = jax.random.gumbel(jax.random.key(42), (N - 1 - 8,), f32)
    g_node = jnp.concatenate(
        [jnp.zeros((8,), f32), g, jnp.zeros((PAD - (N - 1),), f32)]
    ).reshape(ROWS, HALF)
    nidx11 = jnp.asarray(node_index, jnp.int32).reshape(1, 1)
    xlast = x[-1].reshape(2, HALF)
    wrb = Wr[F:2 * F, 0].reshape(2, HALF)
    br2 = br.reshape(1, 1)

    oh, lp = _score(agg2, dinr, w2r, b2.reshape(1, F), wr_ac, g_node,
                    cnt_pad, lastp, srcp, nidx11, xlast, wrb, br2)
    sample_full = oh.reshape(PAD)[:N - 1]
    log_prob = lp.reshape(())
    return (sample_full, log_prob)


# speedup vs baseline: 7.4417x; 1.0017x over previous
import functools

import jax
import jax.numpy as jnp
from jax import lax
from jax.experimental import pallas as pl
from jax.experimental.pallas import tpu as pltpu
from jax.experimental.pallas import tpu_sc as plsc

N = 10000
NP = 10240
E = 160000
F = 256
HALF = 128
NT = 16
EPT = E // NT
CH = 80
NCH = EPT // CH
SL = NP // NT
ALPHA = 1000000.0
BLK = 1000
GRID = N // BLK

_mesh = plsc.VectorSubcoreMesh(core_axis_name="c", subcore_axis_name="s")


def _stats_body(src3, dst3, nidx_h, ones_h, dout_h, din_h, cnt_h, lastp_h,
                srcp_h, srcl, dstl, wvs, ones_v, nv, lastv_s, srcv_s, vbuf,
                hist_sp, cnt_sp, sem_h, sem_w):
    c = lax.axis_index("c")
    s = lax.axis_index("s")
    for j in range(SL // 16):
        vbuf[pl.ds(16 * j, 16)] = jnp.zeros((16,), jnp.float32)
    pltpu.sync_copy(vbuf, hist_sp.at[pl.ds(s * SL, SL)])
    pltpu.sync_copy(src3.at[s], srcl)
    pltpu.sync_copy(ones_h, ones_v)

    @pl.when(c == 1)
    def _():
        pltpu.sync_copy(vbuf, cnt_sp.at[pl.ds(s * SL, SL)])
        pltpu.sync_copy(dst3.at[s], dstl)
        pltpu.sync_copy(nidx_h, nv)

    plsc.subcore_barrier()

    @pl.when(c == 0)
    def _():
        def bk(k, _):
            pltpu.async_copy(ones_v, hist_sp.at[srcl.at[k]], sem_h, add=True)

            @pl.when(k >= 4)
            def _():
                pltpu.make_async_copy(ones_v, hist_sp.at[srcl.at[0]],
                                      sem_h).wait()
            return 0

        lax.fori_loop(0, NCH, bk, 0)
        for _ in range(4):
            pltpu.make_async_copy(ones_v, hist_sp.at[srcl.at[0]], sem_h).wait()

    @pl.when(c == 1)
    def _():
        nvec = nv[...]
        lanes = lax.iota(jnp.int32, 16)
        base = s * EPT

        def bk(k, carry):
            lastv, srcv = carry
            for j in range(CH // 16):
                s16 = srcl[k, pl.ds(16 * j, 16)]
                d16 = dstl[k, pl.ds(16 * j, 16)]
                m = d16 == nvec
                w = jnp.where(m & (s16 < N - 1), 1.0, 0.0).astype(jnp.float32)
                wvs[k, pl.ds(16 * j, 16)] = w
                eid = base + k * CH + 16 * j + lanes
                upd = m & (eid > lastv)
                lastv = jnp.where(upd, eid, lastv)
                srcv = jnp.where(upd, s16, srcv)
            pltpu.async_copy(ones_v, hist_sp.at[dstl.at[k]], sem_h, add=True)
            pltpu.async_copy(wvs.at[k], cnt_sp.at[srcl.at[k]], sem_w, add=True)

            @pl.when(k >= 4)
            def _():
                pltpu.make_async_copy(ones_v, hist_sp.at[dstl.at[0]],
                                      sem_h).wait()
                pltpu.make_async_copy(wvs.at[0], cnt_sp.at[srcl.at[0]],
                                      sem_w).wait()
            return (lastv, srcv)

        init = (jnp.full((16,), -1, jnp.int32), jnp.full((16,), -1, jnp.int32))
        lastv, srcv = lax.fori_loop(0, NCH, bk, init)
        for _ in range(4):
            pltpu.make_async_copy(ones_v, hist_sp.at[dstl.at[0]], sem_h).wait()
            pltpu.make_async_copy(wvs.at[0], cnt_sp.at[srcl.at[0]],
                                  sem_w).wait()
        lastv_s[...] = lastv
        srcv_s[...] = srcv
        pltpu.sync_copy(lastv_s, lastp_h.at[s])
        pltpu.sync_copy(srcv_s, srcp_h.at[s])

    plsc.subcore_barrier()

    @pl.when(c == 0)
    def _():
        pltpu.sync_copy(hist_sp.at[pl.ds(s * SL, SL)], vbuf)
        pltpu.sync_copy(vbuf, dout_h.at[pl.ds(s * SL, SL)])

    @pl.when(c == 1)
    def _():
        pltpu.sync_copy(hist_sp.at[pl.ds(s * SL, SL)], vbuf)
        pltpu.sync_copy(vbuf, din_h.at[pl.ds(s * SL, SL)])
        pltpu.sync_copy(cnt_sp.at[pl.ds(s * SL, SL)], vbuf)
        pltpu.sync_copy(vbuf, cnt_h.at[pl.ds(s * SL, SL)])


_sc_params = pltpu.CompilerParams(use_tc_tiling_on_sc=False)

_stats_call = functools.partial(
    pl.kernel, _stats_body, mesh=_mesh, compiler_params=_sc_params,
    out_type=(
        jax.ShapeDtypeStruct((NP,), jnp.float32),
        jax.ShapeDtypeStruct((NP,), jnp.float32),
        jax.ShapeDtypeStruct((NP,), jnp.float32),
        jax.ShapeDtypeStruct((NT, 16), jnp.int32),
        jax.ShapeDtypeStruct((NT, 16), jnp.int32),
    ),
    scratch_types=[
        pltpu.VMEM((NCH, CH), jnp.int32),
        pltpu.VMEM((NCH, CH), jnp.int32),
        pltpu.VMEM((NCH, CH), jnp.float32),
        pltpu.VMEM((CH,), jnp.float32),
        pltpu.VMEM((16,), jnp.int32),
        pltpu.VMEM((16,), jnp.int32),
        pltpu.VMEM((16,), jnp.int32),
        pltpu.VMEM((SL,), jnp.float32),
        pltpu.VMEM_SHARED((NP,), jnp.float32),
        pltpu.VMEM_SHARED((NP,), jnp.float32),
        pltpu.SemaphoreType.DMA,
        pltpu.SemaphoreType.DMA,
    ],
)


SLAB = 64
CHS = 125
NCHS = EPT // CHS


NBUF = 8


def _spmm_body(feat_h, src3, dst3, z640_h, agg_h, srcl, dstl, *rest):
    rows = rest[:NBUF]
    agg_sp, sem_g, sem_s = rest[NBUF:]
    c = lax.axis_index("c")
    s = lax.axis_index("s")
    pltpu.sync_copy(src3.at[s], srcl)
    pltpu.sync_copy(dst3.at[s], dstl)

    def make_ops(slab):
        fview = feat_h.at[slab]

        def g_start(k, rbuf):
            pltpu.async_copy(fview.at[srcl.at[k]], rbuf, sem_g)

        def g_wait(rbuf):
            pltpu.make_async_copy(fview.at[srcl.at[0]], rbuf, sem_g).wait()

        def s_start(k, rbuf):
            pltpu.async_copy(rbuf, agg_sp.at[dstl.at[k]], sem_s, add=True)

        def s_wait(rbuf):
            pltpu.make_async_copy(rbuf, agg_sp.at[dstl.at[0]], sem_s).wait()

        return g_start, g_wait, s_start, s_wait

    for p in range(2):
        slab = c * 2 + p
        g_start, g_wait, s_start, s_wait = make_ops(slab)
        pltpu.sync_copy(z640_h, agg_sp.at[pl.ds(s * SL, SL)])
        plsc.subcore_barrier()
        for b in range(6):
            g_start(b, rows[b])

        def body_k(i, _):
            kk = NBUF * i
            for b in range(NBUF):
                k = kk + b
                g_wait(rows[b])
                s_start(k, rows[b])

                @pl.when(k >= 2)
                def _():
                    s_wait(rows[(b + 6) % NBUF])

                @pl.when(k + 6 < NCHS)
                def _():
                    g_start(k + 6, rows[(b + 6) % NBUF])
            return 0

        lax.fori_loop(0, NCHS // NBUF, body_k, 0)
        for k in range(NCHS - 2, NCHS):
            s_wait(rows[k % NBUF])
        plsc.subcore_barrier()
        pltpu.sync_copy(agg_sp.at[pl.ds(s * SL, SL)],
                        agg_h.at[slab, pl.ds(s * SL, SL)])
        plsc.subcore_barrier()


_spmm_call = functools.partial(
    pl.kernel, _spmm_body, mesh=_mesh, compiler_params=_sc_params,
    out_type=jax.ShapeDtypeStruct((4, NP, SLAB), jnp.float32),
    scratch_types=[
        pltpu.VMEM((NCHS, CHS), jnp.int32),
        pltpu.VMEM((NCHS, CHS), jnp.int32),
    ] + [pltpu.VMEM((CHS, SLAB), jnp.float32) for _ in range(NBUF)] + [
        pltpu.VMEM_SHARED((NP, SLAB), jnp.float32),
        pltpu.SemaphoreType.DMA,
        pltpu.SemaphoreType.DMA,
    ],
)


BLK2 = 2048
GRID2 = NP // BLK2


def _prep_body(x_ref, dop_ref, dip_ref, feat_ref, doutr_ref, dinr_ref):
    dor = lax.rsqrt(jnp.maximum(dop_ref[...], 1.0))
    dir_ = lax.rsqrt(jnp.maximum(dip_ref[...], 1.0))
    x3 = x_ref[...].reshape(16, HALF, F)
    xf = x3 * dor[:, :, None]
    for j in range(4):
        feat_ref[j] = xf[:, :, SLAB * j:SLAB * (j + 1)].reshape(BLK2, SLAB)
    doutr_ref[...] = dor
    dinr_ref[...] = dir_


def _prep(x, dout, din):
    return pl.pallas_call(
        _prep_body,
        grid=(GRID2,),
        in_specs=[
            pl.BlockSpec((BLK2, F), lambda i: (i, 0)),
            pl.BlockSpec((16, HALF), lambda i: (i, 0)),
            pl.BlockSpec((16, HALF), lambda i: (i, 0)),
        ],
        out_specs=[
            pl.BlockSpec((4, BLK2, SLAB), lambda i: (0, i, 0)),
            pl.BlockSpec((16, HALF), lambda i: (i, 0)),
            pl.BlockSpec((16, HALF), lambda i: (i, 0)),
        ],
        out_shape=[
            jax.ShapeDtypeStruct((4, NP, SLAB), jnp.float32),
            jax.ShapeDtypeStruct((80, HALF), jnp.float32),
            jax.ShapeDtypeStruct((80, HALF), jnp.float32),
        ],
    )(x, dout, din)


def _dense1_body(agg_ref, dinr_ref, doutr_ref, w_ref, b_ref, out_ref):
    dinc = dinr_ref[...][:, :, None]
    z = b_ref[...]
    for j in range(4):
        a = (agg_ref[j].reshape(16, HALF, SLAB) * dinc).reshape(BLK2, SLAB)
        z = z + jnp.dot(a, w_ref[j], preferred_element_type=jnp.float32)
    h = jnp.where(z >= 0, z, 0.01 * z).reshape(16, HALF, F)
    h = h * doutr_ref[...][:, :, None]
    for j in range(4):
        out_ref[j] = h[:, :, SLAB * j:SLAB * (j + 1)].reshape(BLK2, SLAB)


def _dense1(agg, dinr, doutr, w, b):
    return pl.pallas_call(
        _dense1_body,
        grid=(GRID2,),
        in_specs=[
            pl.BlockSpec((4, BLK2, SLAB), lambda i: (0, i, 0)),
            pl.BlockSpec((16, HALF), lambda i: (i, 0)),
            pl.BlockSpec((16, HALF), lambda i: (i, 0)),
            pl.BlockSpec((4, SLAB, F), lambda i: (0, 0, 0)),
            pl.BlockSpec((1, F), lambda i: (0, 0)),
        ],
        out_specs=pl.BlockSpec((4, BLK2, SLAB), lambda i: (0, i, 0)),
        out_shape=jax.ShapeDtypeStruct((4, NP, SLAB), jnp.float32),
    )(agg, dinr, doutr, w, b)


ROWS = 80
PAD = ROWS * HALF


def _score_body(agg_ref, dinr_ref, w_ref, b_ref, wr_ref, g_ref, cnt_ref,
                lastp_ref, srcp_ref, nidx_ref, xlast_ref, wrb_ref, br_ref,
                oh_ref, lp_ref, s_sc, q_sc):
    i = pl.program_id(0)
    dinc = dinr_ref[...][:, :, None]
    z = b_ref[...]
    for j in range(4):
        a = (agg_ref[j].reshape(16, HALF, SLAB) * dinc).reshape(BLK2, SLAB)
        z = z + jnp.dot(a, w_ref[j], preferred_element_type=jnp.float32)
    h = jnp.where(z >= 0, z, 0.01 * z)
    sq = jnp.dot(h, wr_ref[...], preferred_element_type=jnp.float32)
    s_sc[pl.ds(16 * i, 16), :] = sq[:, 0].reshape(16, HALF)
    q_sc[pl.ds(16 * i, 16), :] = sq[:, 1].reshape(16, HALF)

    @pl.when(i == GRID2 - 1)
    def _():
        r = lax.broadcasted_iota(jnp.int32, (ROWS, HALF), 0)
        col = lax.broadcasted_iota(jnp.int32, (ROWS, HALF), 1)
        i2 = r * HALF + col
        valid = (i2 >= 8) & (i2 <= N - 2)
        lastm = jnp.max(lastp_ref[...])
        srcl = jnp.sum(jnp.where(lastp_ref[...] == lastm, srcp_ref[...], 0))
        srcl = jnp.where(lastm < 0, -1, srcl)
        cnt = cnt_ref[...] - jnp.where(i2 == srcl, 1.0, 0.0)
        nm = jnp.where(cnt > 0.5, ALPHA, 0.0)
        nidx = jnp.sum(nidx_ref[...])
        csc = (jnp.sum(jnp.where(i2 == nidx, q_sc[...], 0.0))
               + jnp.sum(xlast_ref[...] * wrb_ref[...]) + jnp.sum(br_ref[...]))
        fd = jnp.where(valid, s_sc[...] + csc + nm, -1e30)
        mx = jnp.max(fd)
        e = jnp.where(valid, jnp.exp(fd - mx), 0.0)
        p = e / jnp.sum(e)
        lp = jnp.log(p)
        t = lp + g_ref[...]
        tm = jnp.max(t)
        idxn = jnp.min(jnp.where(t == tm, i2, jnp.int32(2**30)))
        sel = i2 == idxn
        oh_ref[...] = jnp.where(sel, 1.0, 0.0)
        lpv = jnp.sum(jnp.where(sel, jnp.where(valid, lp, 0.0), 0.0))
        lp_ref[...] = jnp.reshape(lpv, (1, 1))


def _score(agg, dinr, w, b, wr, g_node, cnt, lastp, srcp, nidx, xlast, wrb,
           br2):
    return pl.pallas_call(
        _score_body,
        grid=(GRID2,),
        in_specs=[
            pl.BlockSpec((4, BLK2, SLAB), lambda i: (0, i, 0)),
            pl.BlockSpec((16, HALF), lambda i: (i, 0)),
            pl.BlockSpec((4, SLAB, F), lambda i: (0, 0, 0)),
            pl.BlockSpec((1, F), lambda i: (0, 0)),
            pl.BlockSpec((F, 2), lambda i: (0, 0)),
            pl.BlockSpec((ROWS, HALF), lambda i: (0, 0)),
            pl.BlockSpec((ROWS, HALF), lambda i: (0, 0)),
            pl.BlockSpec((NT, 16), lambda i: (0, 0)),
            pl.BlockSpec((NT, 16), lambda i: (0, 0)),
            pl.BlockSpec((1, 1), lambda i: (0, 0)),
            pl.BlockSpec((2, HALF), lambda i: (0, 0)),
            pl.BlockSpec((2, HALF), lambda i: (0, 0)),
            pl.BlockSpec((1, 1), lambda i: (0, 0)),
        ],
        out_specs=[
            pl.BlockSpec((ROWS, HALF), lambda i: (0, 0)),
            pl.BlockSpec((1, 1), lambda i: (0, 0)),
        ],
        out_shape=[
            jax.ShapeDtypeStruct((ROWS, HALF), jnp.float32),
            jax.ShapeDtypeStruct((1, 1), jnp.float32),
        ],
        scratch_shapes=[
            pltpu.VMEM((ROWS, HALF), jnp.float32),
            pltpu.VMEM((ROWS, HALF), jnp.float32),
        ],
    )(agg, dinr, w, b, wr, g_node, cnt, lastp, srcp, nidx, xlast, wrb, br2)


def kernel(x, edge_index, node_index, edge_set, W1, b1, W2, b2, Wr, br):
    f32 = jnp.float32
    src = edge_index[0].astype(jnp.int32)
    dst = edge_index[1].astype(jnp.int32)
    src3 = src.reshape(NT, NCH, CH)
    dst3 = dst.reshape(NT, NCH, CH)
    src3s = src.reshape(NT, NCHS, CHS)
    dst3s = dst.reshape(NT, NCHS, CHS)
    nidx16 = jnp.full((16,), node_index, jnp.int32)
    ones80 = jnp.ones((CH,), f32)

    dout_h, din_h, cnt_h, lastp, srcp = _stats_call()(
        src3, dst3, nidx16, ones80)

    feat, doutr, dinr = _prep(x, dout_h.reshape(ROWS, HALF),
                              din_h.reshape(ROWS, HALF))
    w1r = W1.reshape(4, SLAB, F)
    w2r = W2.reshape(4, SLAB, F)
    z640 = jnp.zeros((SL, SLAB), f32)
    agg1 = _spmm_call()(feat, src3s, dst3s, z640)
    feat2 = _dense1(agg1, dinr, doutr, w1r, b1.reshape(1, F))
    agg2 = _spmm_call()(feat2, src3s, dst3s, z640)
    wr_ac = jnp.concatenate([Wr[0:F], Wr[2 * F:3 * F]], axis=1)
    cnt_pad = cnt_h.reshape(ROWS, HALF)
    g = jax.random.gumbel(jax.random.key(42), (N - 1 - 8,), f32)
    g_node = jnp.concatenate(
        [jnp.zeros((8,), f32), g, jnp.zeros((PAD - (N - 1),), f32)]
    ).reshape(ROWS, HALF)
    nidx11 = jnp.asarray(node_index, jnp.int32).reshape(1, 1)
    xlast = x[-1].reshape(2, HALF)
    wrb = Wr[F:2 * F, 0].reshape(2, HALF)
    br2 = br.reshape(1, 1)

    oh, lp = _score(agg2, dinr, w2r, b2.reshape(1, F), wr_ac, g_node,
                    cnt_pad, lastp, srcp, nidx11, xlast, wrb, br2)
    sample_full = oh.reshape(PAD)[:N - 1]
    log_prob = lp.reshape(())
    return (sample_full, log_prob)
